# baseline (speedup 1.0000x reference)
import jax
import jax.numpy as jnp
from jax import lax
from jax.experimental import pallas as pl
from jax.experimental.pallas import tpu as pltpu
from jax.experimental.pallas import tpu_sc as plsc

D = 128
NEIGH = 32
NC = 2
NS = 16
B_PAD = 10240
HALF = B_PAD // NC
C = 4
E = C * NEIGH
U = 32
UC = U // C
UI = U * NEIGH
NUNIT = HALF // U
NBUF = 4
NCN = 64
NNOD = HALF // NCN


def _tc1_body(x_ref, w_ref, b_ref, o_ref):
    acc = jnp.dot(x_ref[...], w_ref[...], preferred_element_type=jnp.float32)
    o_ref[...] = jnp.maximum(acc + b_ref[...], 0.0)


def _transform_table(features, W_dense, b_dense):
    n = features.shape[0]
    blk = 1000
    grid = n // blk
    return pl.pallas_call(
        _tc1_body,
        grid=(grid,),
        in_specs=[
            pl.BlockSpec((blk, D), lambda i: (i, 0)),
            pl.BlockSpec((D, D), lambda i: (0, 0)),
            pl.BlockSpec((1, D), lambda i: (0, 0)),
        ],
        out_specs=pl.BlockSpec((blk, D), lambda i: (i, 0)),
        out_shape=jax.ShapeDtypeStruct((n, D), jnp.float32),
    )(features, W_dense, b_dense.reshape(1, D))


def _sc_body(t_hbm, feat_hbm, neigh_hbm, node_hbm, pooled_hbm, nodef_hbm,
             idxb_v, b0, b1, b2, b3, stg0, stg1, nidx_v, cnt,
             sneigh_sp,
             s0, s1, s2, s3, os0, os1, psem):
    bufs = (b0, b1, b2, b3)
    sems = (s0, s1, s2, s3)
    stgs = (stg0, stg1)
    osems = (os0, os1)
    cid = lax.axis_index("c")
    sid = lax.axis_index("s")
    scbase = cid * HALF

    @pl.when(sid == 0)
    def _():
        cnt[0] = 0
        cnt[1] = 0

    NSTRIPE = HALF * NEIGH // NS
    pltpu.async_copy(
        neigh_hbm.at[pl.ds(scbase * NEIGH + sid * NSTRIPE, NSTRIPE)],
        sneigh_sp.at[pl.ds(sid * NSTRIPE, NSTRIPE)], psem)
    pltpu.make_async_copy(
        neigh_hbm.at[pl.ds(0, NSTRIPE)],
        sneigh_sp.at[pl.ds(0, NSTRIPE)], psem).wait()
    plsc.subcore_barrier()

    def node_take():
        return plsc.fetch_and_add(cnt.at[1], 1, subcore_id=0)

    def node_body(j):
        off = scbase + j * NCN
        pltpu.sync_copy(node_hbm.at[pl.ds(off, NCN)], nidx_v)
        pltpu.async_copy(feat_hbm.at[nidx_v], b0.at[pl.ds(0, NCN)], s0).wait()
        pltpu.sync_copy(b0.at[pl.ds(0, NCN)], nodef_hbm.at[pl.ds(off, NCN)])
        return node_take()

    with jax.named_scope("node_gather"):
        lax.while_loop(lambda j: j < NNOD, node_body, node_take())

    def pool_take():
        return plsc.fetch_and_add(cnt.at[0], 1, subcore_id=0)

    def load_idx(u, pp):
        pltpu.sync_copy(sneigh_sp.at[pl.ds(u * UI, UI)],
                        idxb_v.at[pl.ds(pp * UI, UI)])

    def fire(b, pp):
        pltpu.async_copy(
            t_hbm.at[idxb_v.at[pl.ds(pp * UI + b * E, E)]],
            bufs[b % NBUF], sems[b % NBUF])

    def wait_buf(b):
        pltpu.make_async_copy(t_hbm.at[idxb_v.at[pl.ds(0, E)]],
                              bufs[b % NBUF], sems[b % NBUF]).wait()

    def accumulate(rows_v, stg, row0):
        def nbody(n, accs):
            new = []
            for c in range(C):
                for d in range(D // 16):
                    new.append(accs[c * (D // 16) + d]
                               + rows_v[c * NEIGH + n, pl.ds(d * 16, 16)])
            return tuple(new)
        init = tuple(jnp.zeros((16,), jnp.float32) for _ in range(C * (D // 16)))
        accs = lax.fori_loop(0, NEIGH, nbody, init)
        for c in range(C):
            for d in range(D // 16):
                stg[row0 + c, pl.ds(d * 16, 16)] = (
                    accs[c * (D // 16) + d] * (1.0 / NEIGH))

    def drain_stg(pp):
        pltpu.make_async_copy(stgs[pp], pooled_hbm.at[pl.ds(scbase, U)],
                              osems[pp]).wait()

    def run_unit(u, nxt, pp):
        for b in range(NBUF - 1):
            fire(b, pp)
        @pl.when(nxt < NUNIT)
        def _():
            load_idx(nxt, 1 - pp)
        for b in range(UC):
            if b + NBUF - 1 < UC:
                fire(b + NBUF - 1, pp)
            wait_buf(b)
            accumulate(bufs[b % NBUF], stgs[pp], b * C)
        pltpu.async_copy(stgs[pp],
                         pooled_hbm.at[pl.ds(scbase + u * U, U)],
                         osems[pp])

    with jax.named_scope("pool_loop"):
        prev0 = pool_take()

        @pl.when(prev0 < NUNIT)
        def _():
            load_idx(prev0, 0)

        def body(carry):
            prev, p, k = carry
            nxt = pool_take()
            for pp in range(2):
                @pl.when(p == pp)
                def _(pp=pp):
                    @pl.when(k >= 2)
                    def _():
                        drain_stg(pp)
                    run_unit(prev, nxt, pp)
            return (nxt, 1 - p, k + 1)

        prev, p, k = lax.while_loop(lambda c: c[0] < NUNIT, body,
                                    (prev0, jnp.int32(0), jnp.int32(0)))

    with jax.named_scope("pooled_drain"):
        @pl.when(k >= 1)
        def _():
            drain_stg(0)

        @pl.when(k >= 2)
        def _():
            drain_stg(1)


def _sc_gather_pool(T, features, neigh_flat, node_flat):
    mesh = plsc.VectorSubcoreMesh(core_axis_name="c", subcore_axis_name="s")
    return pl.kernel(
        _sc_body,
        compiler_params=pltpu.CompilerParams(needs_layout_passes=False),
        out_type=(
            jax.ShapeDtypeStruct((B_PAD, D), jnp.float32),
            jax.ShapeDtypeStruct((B_PAD, D), jnp.float32),
        ),
        mesh=mesh,
        scratch_types=[
            pltpu.VMEM((2 * UI,), jnp.int32),
            pltpu.VMEM((E, D), jnp.float32),
            pltpu.VMEM((E, D), jnp.float32),
            pltpu.VMEM((E, D), jnp.float32),
            pltpu.VMEM((E, D), jnp.float32),
            pltpu.VMEM((U, D), jnp.float32),
            pltpu.VMEM((U, D), jnp.float32),
            pltpu.VMEM((NCN,), jnp.int32),
            pltpu.SMEM((2,), jnp.int32),
            pltpu.VMEM_SHARED((HALF * NEIGH,), jnp.int32),
            pltpu.SemaphoreType.DMA,
            pltpu.SemaphoreType.DMA,
            pltpu.SemaphoreType.DMA,
            pltpu.SemaphoreType.DMA,
            pltpu.SemaphoreType.DMA,
            pltpu.SemaphoreType.DMA,
            pltpu.SemaphoreType.DMA,
        ],
    )(T, features, neigh_flat, node_flat)


def _tc2_body(nf_ref, pv_ref, w1_ref, w2_ref, o_ref):
    acc = jnp.dot(nf_ref[...], w1_ref[...], preferred_element_type=jnp.float32)
    acc = acc + jnp.dot(pv_ref[...], w2_ref[...], preferred_element_type=jnp.float32)
    o_ref[...] = jnp.maximum(acc, 0.0)


def _final_matmul(nodef, pooled, w1, w2, b):
    blk = 1000
    grid = b // blk
    return pl.pallas_call(
        _tc2_body,
        grid=(grid,),
        in_specs=[
            pl.BlockSpec((blk, D), lambda i: (i, 0)),
            pl.BlockSpec((blk, D), lambda i: (i, 0)),
            pl.BlockSpec((D, D), lambda i: (0, 0)),
            pl.BlockSpec((D, D), lambda i: (0, 0)),
        ],
        out_specs=pl.BlockSpec((blk, D), lambda i: (i, 0)),
        out_shape=jax.ShapeDtypeStruct((b, D), jnp.float32),
    )(nodef, pooled, w1, w2)


def kernel(features, node, neighbours, W_dense, b_dense, neigh_weights):
    b = node.shape[0]
    pad = B_PAD - b
    node_flat = node.reshape(b).astype(jnp.int32)
    node_p = jnp.pad(node_flat, (0, pad))
    neigh_p = jnp.pad(neighbours.astype(jnp.int32), ((0, pad), (0, 0)))
    neigh_flat = neigh_p.reshape(B_PAD * NEIGH)

    T = _transform_table(features, W_dense, b_dense)
    pooled, nodef = _sc_gather_pool(T, features, neigh_flat, node_p)
    out = _final_matmul(nodef, pooled, neigh_weights[:D], neigh_weights[D:], b)
    return out
